# baseline (speedup 1.0000x reference)
import functools

import jax
import jax.numpy as jnp
from jax import lax
from jax.experimental import pallas as pl
from jax.experimental.pallas import tpu as pltpu
from jax.experimental.pallas import tpu_sc as plsc

N = 10000
E = 320000
D_IN = 128
D_H = 64
D_OUT = 32

NC = 2
NS = 16
CH = 128
KD = 80
KE = 160
N_PAD = 10240
RPT = N_PAD // NS
NBUF = 8

_mesh = plsc.VectorSubcoreMesh(core_axis_name="c", subcore_axis_name="s")


def _sc_degree(dsti, zeros1):
    @functools.partial(
        pl.kernel,
        mesh=_mesh,
        compiler_params=pltpu.CompilerParams(use_tc_tiling_on_sc=False),
        out_type=jax.ShapeDtypeStruct((NC, N_PAD), jnp.float32),
        scratch_types=[
            pltpu.VMEM((KD, CH), jnp.int32),
            pltpu.VMEM((CH,), jnp.float32),
            pltpu.VMEM_SHARED((N_PAD,), jnp.float32),
        ],
    )
    def k(dst_h, z_h, out, dst_v, ones_v, dacc):
        cid = lax.axis_index("c")
        sid = lax.axis_index("s")
        pltpu.sync_copy(dst_h.at[cid, sid], dst_v)
        for t in range(CH // 16):
            ones_v[pl.ds(16 * t, 16)] = jnp.full((16,), 1.0, jnp.float32)
        r0 = sid * RPT
        pltpu.sync_copy(z_h.at[pl.ds(r0, RPT)], dacc.at[pl.ds(r0, RPT)])
        plsc.subcore_barrier()

        def body(j, carry):
            pltpu.sync_copy(ones_v, dacc.at[dst_v.at[j]], add=True)
            return carry

        lax.fori_loop(0, KD, body, 0)
        plsc.subcore_barrier()
        pltpu.sync_copy(dacc.at[pl.ds(r0, RPT)], out.at[cid, pl.ds(r0, RPT)])

    return k(dsti, zeros1)


def _sc_edge_pass(y, srci, dsti, d, bias, dt):
    dc = dt // NC

    @functools.partial(
        pl.kernel,
        mesh=_mesh,
        compiler_params=pltpu.CompilerParams(use_tc_tiling_on_sc=False),
        out_type=jax.ShapeDtypeStruct((N_PAD, dt), jnp.float32),
        scratch_types=[
            pltpu.VMEM((KE, CH), jnp.int32),
            pltpu.VMEM((KE, CH), jnp.int32),
            [pltpu.VMEM((CH, dc), jnp.float32) for _ in range(NBUF)],
            pltpu.VMEM((RPT,), jnp.float32),
            pltpu.VMEM((dc,), jnp.float32),
            pltpu.VMEM_SHARED((N_PAD, dc), jnp.float32),
            pltpu.VMEM_SHARED((N_PAD, dc), jnp.float32),
            [pltpu.SemaphoreType.DMA for _ in range(NBUF)],
        ],
    )
    def k(y_h, src_h, dst_h, d_h, b_h, out, src_v, dst_v, rows,
          d_v, b_v, acc, y_s, sems):
        cid = lax.axis_index("c")
        sid = lax.axis_index("s")
        c0 = cid * dc
        pltpu.sync_copy(src_h.at[sid], src_v)
        pltpu.sync_copy(dst_h.at[sid], dst_v)
        r0 = sid * RPT
        def zfill(i, carry):
            rows[0][i // (dc // 16), pl.ds((i % (dc // 16)) * 16, 16)] = (
                jnp.zeros((16,), jnp.float32))
            return carry
        lax.fori_loop(0, CH * dc // 16, zfill, 0)
        for t in range(RPT // CH):
            pltpu.sync_copy(rows[0], acc.at[pl.ds(r0 + t * CH, CH)])
        pltpu.sync_copy(y_h.at[pl.ds(r0, RPT), pl.ds(c0, dc)],
                        y_s.at[pl.ds(r0, RPT)])
        pltpu.sync_copy(d_h.at[pl.ds(r0, RPT)], d_v)
        pltpu.sync_copy(b_h.at[pl.ds(c0, dc)], b_v)
        plsc.subcore_barrier()

        for b in range(NBUF):
            pltpu.make_async_copy(y_s.at[src_v.at[b]], rows[b], sems[b]).start()

        def body(g, carry):
            j = NBUF * g
            for b in range(NBUF):
                pltpu.make_async_copy(
                    y_s.at[src_v.at[j + b]], rows[b], sems[b]).wait()
                pltpu.sync_copy(rows[b], acc.at[dst_v.at[j + b]], add=True)
                pltpu.make_async_copy(
                    y_s.at[src_v.at[j + b + NBUF]], rows[b], sems[b]).start()
            return carry

        lax.fori_loop(0, KE // NBUF - 1, body, 0)
        for b in range(NBUF):
            j = KE - NBUF + b
            pltpu.make_async_copy(y_s.at[src_v.at[j]], rows[b], sems[b]).wait()
            pltpu.sync_copy(rows[b], acc.at[dst_v.at[j]], add=True)

        plsc.subcore_barrier()
        for t in range(RPT // CH):
            ra = r0 + t * CH
            pltpu.sync_copy(acc.at[pl.ds(ra, CH)], rows[0])
            pltpu.sync_copy(y_s.at[pl.ds(ra, CH)], rows[1])

            def erow(g, carry):
                rg = g * 16
                d16 = d_v[pl.ds(t * CH + rg, 16)]
                for i in range(16):
                    dvec = jnp.full((16,), d16[i], jnp.float32)
                    for cb in range(dc // 16):
                        cs = pl.ds(cb * 16, 16)
                        v = (rows[0][rg + i, cs] + rows[1][rg + i, cs]) * dvec
                        rows[0][rg + i, cs] = jnp.maximum(v + b_v[cs], 0.0)
                return carry

            lax.fori_loop(0, CH // 16, erow, 0)
            pltpu.sync_copy(rows[0], out.at[pl.ds(ra, CH), pl.ds(c0, dc)])

    return k(y, srci, dsti, d, bias)


def _tc_mm(x_pad, W1):
    def body(x_ref, w_ref, o_ref):
        o_ref[...] = jnp.dot(x_ref[...], w_ref[...],
                             preferred_element_type=jnp.float32)

    return pl.pallas_call(
        body,
        out_shape=jax.ShapeDtypeStruct((N_PAD, D_H), jnp.float32),
    )(x_pad, W1)


def _tc_scale(deg2, xw):
    def body(deg_ref, xw_ref, d_ref, y_ref):
        deg = deg_ref[0] + deg_ref[1] + 1.0
        dvec = lax.rsqrt(deg)
        d_ref[...] = dvec
        y_ref[...] = xw_ref[...] * dvec[:, None]

    return pl.pallas_call(
        body,
        out_shape=[
            jax.ShapeDtypeStruct((N_PAD,), jnp.float32),
            jax.ShapeDtypeStruct((N_PAD, D_H), jnp.float32),
        ],
    )(deg2, xw)


def _tc_mid(h, d, W2):
    def body(h_ref, d_ref, w_ref, y2_ref):
        y2_ref[...] = jnp.dot(
            h_ref[...], w_ref[...],
            preferred_element_type=jnp.float32) * d_ref[...][:, None]

    return pl.pallas_call(
        body,
        out_shape=jax.ShapeDtypeStruct((N_PAD, D_OUT), jnp.float32),
    )(h, d, W2)


def kernel(x, edge_index, W1, b1, W2, b2):
    src = edge_index[0]
    dst = edge_index[1]
    tot = NC * NS * KD * CH
    dst_deg = jnp.concatenate(
        [dst, jnp.full((tot - E,), N, jnp.int32)]).reshape(NC, NS, KD, CH)
    tot_e = NS * KE * CH
    pad_e = tot_e - E
    srci = jnp.concatenate(
        [src, jnp.zeros((pad_e,), jnp.int32)]).reshape(NS, KE, CH)
    dsti = jnp.concatenate(
        [dst, jnp.full((pad_e,), N, jnp.int32)]).reshape(NS, KE, CH)

    x_pad = jnp.pad(x, ((0, N_PAD - N), (0, 0)))
    z1 = jnp.zeros((N_PAD,), jnp.float32)

    deg2 = _sc_degree(dst_deg, z1)
    xw = _tc_mm(x_pad, W1)
    d, y1 = _tc_scale(deg2, xw)
    h = _sc_edge_pass(y1, srci, dsti, d, b1, D_H)
    y2 = _tc_mid(h, d, W2)
    out = _sc_edge_pass(y2, srci, dsti, d, b2, D_OUT)
    return out[:N]


# speedup vs baseline: 1.0383x; 1.0383x over previous
import functools

import jax
import jax.numpy as jnp
from jax import lax
from jax.experimental import pallas as pl
from jax.experimental.pallas import tpu as pltpu
from jax.experimental.pallas import tpu_sc as plsc

N = 10000
E = 320000
D_IN = 128
D_H = 64
D_OUT = 32

NC = 2
NS = 16
CH = 128
KD = 80
KE = 160
N_PAD = 10240
RPT = N_PAD // NS
NBUF = 8

_mesh = plsc.VectorSubcoreMesh(core_axis_name="c", subcore_axis_name="s")


def _sc_degree(dsti, zeros1):
    @functools.partial(
        pl.kernel,
        mesh=_mesh,
        compiler_params=pltpu.CompilerParams(use_tc_tiling_on_sc=False),
        out_type=jax.ShapeDtypeStruct((NC, N_PAD), jnp.float32),
        scratch_types=[
            pltpu.VMEM((KD, CH), jnp.int32),
            pltpu.VMEM((CH,), jnp.float32),
            pltpu.VMEM_SHARED((N_PAD,), jnp.float32),
        ],
    )
    def k(dst_h, z_h, out, dst_v, ones_v, dacc):
        cid = lax.axis_index("c")
        sid = lax.axis_index("s")
        pltpu.sync_copy(dst_h.at[cid, sid], dst_v)
        for t in range(CH // 16):
            ones_v[pl.ds(16 * t, 16)] = jnp.full((16,), 1.0, jnp.float32)
        r0 = sid * RPT
        pltpu.sync_copy(z_h.at[pl.ds(r0, RPT)], dacc.at[pl.ds(r0, RPT)])
        plsc.subcore_barrier()

        def body(j, carry):
            pltpu.sync_copy(ones_v, dacc.at[dst_v.at[j]], add=True)
            return carry

        lax.fori_loop(0, KD, body, 0)
        plsc.subcore_barrier()
        pltpu.sync_copy(dacc.at[pl.ds(r0, RPT)], out.at[cid, pl.ds(r0, RPT)])

    return k(dsti, zeros1)


def _sc_edge_pass(y, srci, dsti, zeros2, dt):
    dc = dt // NC

    @functools.partial(
        pl.kernel,
        mesh=_mesh,
        compiler_params=pltpu.CompilerParams(use_tc_tiling_on_sc=False),
        out_type=jax.ShapeDtypeStruct((N_PAD, dt), jnp.float32),
        scratch_types=[
            pltpu.VMEM((KE, CH), jnp.int32),
            pltpu.VMEM((KE, CH), jnp.int32),
            [pltpu.VMEM((CH, dc), jnp.float32) for _ in range(NBUF)],
            pltpu.VMEM_SHARED((N_PAD, dc), jnp.float32),
            pltpu.VMEM_SHARED((N_PAD, dc), jnp.float32),
            [pltpu.SemaphoreType.DMA for _ in range(NBUF)],
        ],
    )
    def k(y_h, src_h, dst_h, z_h, out, src_v, dst_v, rows, acc, y_s, sems):
        cid = lax.axis_index("c")
        sid = lax.axis_index("s")
        c0 = cid * dc
        pltpu.sync_copy(src_h.at[sid], src_v)
        pltpu.sync_copy(dst_h.at[sid], dst_v)
        r0 = sid * RPT
        pltpu.sync_copy(z_h.at[pl.ds(r0, RPT)], acc.at[pl.ds(r0, RPT)])
        pltpu.sync_copy(y_h.at[pl.ds(r0, RPT), pl.ds(c0, dc)],
                        y_s.at[pl.ds(r0, RPT)])
        plsc.subcore_barrier()

        for b in range(NBUF):
            pltpu.make_async_copy(y_s.at[src_v.at[b]], rows[b], sems[b]).start()

        def body(g, carry):
            j = NBUF * g
            for b in range(NBUF):
                pltpu.make_async_copy(
                    y_s.at[src_v.at[j + b]], rows[b], sems[b]).wait()
                pltpu.sync_copy(rows[b], acc.at[dst_v.at[j + b]], add=True)
                pltpu.make_async_copy(
                    y_s.at[src_v.at[j + b + NBUF]], rows[b], sems[b]).start()
            return carry

        lax.fori_loop(0, KE // NBUF - 1, body, 0)
        for b in range(NBUF):
            j = KE - NBUF + b
            pltpu.make_async_copy(y_s.at[src_v.at[j]], rows[b], sems[b]).wait()
            pltpu.sync_copy(rows[b], acc.at[dst_v.at[j]], add=True)

        plsc.subcore_barrier()
        pltpu.sync_copy(acc.at[pl.ds(r0, RPT)],
                        out.at[pl.ds(r0, RPT), pl.ds(c0, dc)])

    return k(y, srci, dsti, zeros2)


def _tc_mm(x_pad, W1):
    def body(x_ref, w_ref, o_ref):
        o_ref[...] = jnp.dot(x_ref[...], w_ref[...],
                             preferred_element_type=jnp.float32)

    return pl.pallas_call(
        body,
        out_shape=jax.ShapeDtypeStruct((N_PAD, D_H), jnp.float32),
    )(x_pad, W1)


def _tc_scale(deg2, xw):
    def body(deg_ref, xw_ref, d_ref, y_ref):
        deg = deg_ref[0] + deg_ref[1] + 1.0
        dvec = lax.rsqrt(deg)
        d_ref[...] = dvec
        y_ref[...] = xw_ref[...] * dvec[:, None]

    return pl.pallas_call(
        body,
        out_shape=[
            jax.ShapeDtypeStruct((N_PAD,), jnp.float32),
            jax.ShapeDtypeStruct((N_PAD, D_H), jnp.float32),
        ],
    )(deg2, xw)


def _tc_mid(acc1, y1, d, b1, W2):
    def body(acc_ref, y_ref, d_ref, b_ref, w_ref, y2_ref):
        dcol = d_ref[...][:, None]
        h = jnp.maximum((acc_ref[...] + y_ref[...]) * dcol + b_ref[...], 0.0)
        y2_ref[...] = jnp.dot(
            h, w_ref[...], preferred_element_type=jnp.float32) * dcol

    return pl.pallas_call(
        body,
        out_shape=jax.ShapeDtypeStruct((N_PAD, D_OUT), jnp.float32),
    )(acc1, y1, d, b1.reshape(1, D_H), W2)


def _tc_final(acc2, y2, d, b2):
    def body(acc_ref, y_ref, d_ref, b_ref, o_ref):
        dcol = d_ref[...][:, None]
        o_ref[...] = jnp.maximum(
            (acc_ref[...] + y_ref[...]) * dcol + b_ref[...], 0.0)[:N]

    return pl.pallas_call(
        body,
        out_shape=jax.ShapeDtypeStruct((N, D_OUT), jnp.float32),
    )(acc2, y2, d, b2.reshape(1, D_OUT))


def kernel(x, edge_index, W1, b1, W2, b2):
    src = edge_index[0]
    dst = edge_index[1]
    tot = NC * NS * KD * CH
    dst_deg = jnp.concatenate(
        [dst, jnp.full((tot - E,), N, jnp.int32)]).reshape(NC, NS, KD, CH)
    tot_e = NS * KE * CH
    pad_e = tot_e - E
    srci = jnp.concatenate(
        [src, jnp.zeros((pad_e,), jnp.int32)]).reshape(NS, KE, CH)
    dsti = jnp.concatenate(
        [dst, jnp.full((pad_e,), N, jnp.int32)]).reshape(NS, KE, CH)

    x_pad = jnp.pad(x, ((0, N_PAD - N), (0, 0)))
    z1 = jnp.zeros((N_PAD,), jnp.float32)
    zH = jnp.zeros((N_PAD, D_H // NC), jnp.float32)
    zO = jnp.zeros((N_PAD, D_OUT // NC), jnp.float32)

    deg2 = _sc_degree(dst_deg, z1)
    xw = _tc_mm(x_pad, W1)
    d, y1 = _tc_scale(deg2, xw)
    s1 = _sc_edge_pass(y1, srci, dsti, zH, D_H)
    y2 = _tc_mid(s1, y1, d, b1, W2)
    s2 = _sc_edge_pass(y2, srci, dsti, zO, D_OUT)
    return _tc_final(s2, y2, d, b2)


# speedup vs baseline: 1.0411x; 1.0027x over previous
import functools

import jax
import jax.numpy as jnp
from jax import lax
from jax.experimental import pallas as pl
from jax.experimental.pallas import tpu as pltpu
from jax.experimental.pallas import tpu_sc as plsc

N = 10000
E = 320000
D_IN = 128
D_H = 64
D_OUT = 32

NC = 2
NS = 16
CH = 128
KD = 80
KE = 160
N_PAD = 10240
RPT = N_PAD // NS
NBUF = 4

_mesh = plsc.VectorSubcoreMesh(core_axis_name="c", subcore_axis_name="s")


def _sc_degree(dsti, zeros1):
    @functools.partial(
        pl.kernel,
        mesh=_mesh,
        compiler_params=pltpu.CompilerParams(use_tc_tiling_on_sc=False),
        out_type=jax.ShapeDtypeStruct((NC, N_PAD), jnp.float32),
        scratch_types=[
            pltpu.VMEM((KD, CH), jnp.int32),
            pltpu.VMEM((CH,), jnp.float32),
            pltpu.VMEM_SHARED((N_PAD,), jnp.float32),
        ],
    )
    def k(dst_h, z_h, out, dst_v, ones_v, dacc):
        cid = lax.axis_index("c")
        sid = lax.axis_index("s")
        pltpu.sync_copy(dst_h.at[cid, sid], dst_v)
        for t in range(CH // 16):
            ones_v[pl.ds(16 * t, 16)] = jnp.full((16,), 1.0, jnp.float32)
        r0 = sid * RPT
        pltpu.sync_copy(z_h.at[pl.ds(r0, RPT)], dacc.at[pl.ds(r0, RPT)])
        plsc.subcore_barrier()

        def body(j, carry):
            pltpu.sync_copy(ones_v, dacc.at[dst_v.at[j]], add=True)
            return carry

        lax.fori_loop(0, KD, body, 0)
        plsc.subcore_barrier()
        pltpu.sync_copy(dacc.at[pl.ds(r0, RPT)], out.at[cid, pl.ds(r0, RPT)])

    return k(dsti, zeros1)


def _sc_edge_pass(y, srci, dsti, zeros2, dt):
    dc = dt // NC

    @functools.partial(
        pl.kernel,
        mesh=_mesh,
        compiler_params=pltpu.CompilerParams(use_tc_tiling_on_sc=False),
        out_type=jax.ShapeDtypeStruct((N_PAD, dt), jnp.float32),
        scratch_types=[
            pltpu.VMEM((KE, CH), jnp.int32),
            pltpu.VMEM((KE, CH), jnp.int32),
            [pltpu.VMEM((CH, dc), jnp.float32) for _ in range(NBUF)],
            pltpu.VMEM_SHARED((N_PAD, dc), jnp.float32),
            pltpu.VMEM_SHARED((N_PAD, dc), jnp.float32),
            [pltpu.SemaphoreType.DMA for _ in range(NBUF)],
        ],
    )
    def k(y_h, src_h, dst_h, z_h, out, src_v, dst_v, rows, acc, y_s, sems):
        cid = lax.axis_index("c")
        sid = lax.axis_index("s")
        c0 = cid * dc
        pltpu.sync_copy(src_h.at[sid], src_v)
        pltpu.sync_copy(dst_h.at[sid], dst_v)
        r0 = sid * RPT
        pltpu.sync_copy(z_h.at[pl.ds(r0, RPT)], acc.at[pl.ds(r0, RPT)])
        pltpu.sync_copy(y_h.at[pl.ds(r0, RPT), pl.ds(c0, dc)],
                        y_s.at[pl.ds(r0, RPT)])
        plsc.subcore_barrier()

        for b in range(NBUF):
            pltpu.make_async_copy(y_s.at[src_v.at[b]], rows[b], sems[b]).start()

        def body(g, carry):
            j = NBUF * g
            for b in range(NBUF):
                pltpu.make_async_copy(
                    y_s.at[src_v.at[j + b]], rows[b], sems[b]).wait()
                pltpu.sync_copy(rows[b], acc.at[dst_v.at[j + b]], add=True)
                pltpu.make_async_copy(
                    y_s.at[src_v.at[j + b + NBUF]], rows[b], sems[b]).start()
            return carry

        lax.fori_loop(0, KE // NBUF - 1, body, 0)
        for b in range(NBUF):
            j = KE - NBUF + b
            pltpu.make_async_copy(y_s.at[src_v.at[j]], rows[b], sems[b]).wait()
            pltpu.sync_copy(rows[b], acc.at[dst_v.at[j]], add=True)

        plsc.subcore_barrier()
        pltpu.sync_copy(acc.at[pl.ds(r0, RPT)],
                        out.at[pl.ds(r0, RPT), pl.ds(c0, dc)])

    return k(y, srci, dsti, zeros2)


def _tc_mm(x_pad, W1):
    def body(x_ref, w_ref, o_ref):
        o_ref[...] = jnp.dot(x_ref[...], w_ref[...],
                             preferred_element_type=jnp.float32)

    return pl.pallas_call(
        body,
        out_shape=jax.ShapeDtypeStruct((N_PAD, D_H), jnp.float32),
    )(x_pad, W1)


def _tc_scale(deg2, xw):
    def body(deg_ref, xw_ref, d_ref, y_ref):
        deg = deg_ref[0] + deg_ref[1] + 1.0
        dvec = lax.rsqrt(deg)
        d_ref[...] = dvec
        y_ref[...] = xw_ref[...] * dvec[:, None]

    return pl.pallas_call(
        body,
        out_shape=[
            jax.ShapeDtypeStruct((N_PAD,), jnp.float32),
            jax.ShapeDtypeStruct((N_PAD, D_H), jnp.float32),
        ],
    )(deg2, xw)


def _tc_mid(acc1, y1, d, b1, W2):
    def body(acc_ref, y_ref, d_ref, b_ref, w_ref, y2_ref):
        dcol = d_ref[...][:, None]
        h = jnp.maximum((acc_ref[...] + y_ref[...]) * dcol + b_ref[...], 0.0)
        y2_ref[...] = jnp.dot(
            h, w_ref[...], preferred_element_type=jnp.float32) * dcol

    return pl.pallas_call(
        body,
        out_shape=jax.ShapeDtypeStruct((N_PAD, D_OUT), jnp.float32),
    )(acc1, y1, d, b1.reshape(1, D_H), W2)


def _tc_final(acc2, y2, d, b2):
    def body(acc_ref, y_ref, d_ref, b_ref, o_ref):
        dcol = d_ref[...][:, None]
        o_ref[...] = jnp.maximum(
            (acc_ref[...] + y_ref[...]) * dcol + b_ref[...], 0.0)[:N]

    return pl.pallas_call(
        body,
        out_shape=jax.ShapeDtypeStruct((N, D_OUT), jnp.float32),
    )(acc2, y2, d, b2.reshape(1, D_OUT))


def kernel(x, edge_index, W1, b1, W2, b2):
    src = edge_index[0]
    dst = edge_index[1]
    tot = NC * NS * KD * CH
    dst_deg = jnp.concatenate(
        [dst, jnp.full((tot - E,), N, jnp.int32)]).reshape(NC, NS, KD, CH)
    tot_e = NS * KE * CH
    pad_e = tot_e - E
    srci = jnp.concatenate(
        [src, jnp.zeros((pad_e,), jnp.int32)]).reshape(NS, KE, CH)
    dsti = jnp.concatenate(
        [dst, jnp.full((pad_e,), N, jnp.int32)]).reshape(NS, KE, CH)

    x_pad = jnp.pad(x, ((0, N_PAD - N), (0, 0)))
    z1 = jnp.zeros((N_PAD,), jnp.float32)
    zH = jnp.zeros((N_PAD, D_H // NC), jnp.float32)
    zO = jnp.zeros((N_PAD, D_OUT // NC), jnp.float32)

    deg2 = _sc_degree(dst_deg, z1)
    xw = _tc_mm(x_pad, W1)
    d, y1 = _tc_scale(deg2, xw)
    s1 = _sc_edge_pass(y1, srci, dsti, zH, D_H)
    y2 = _tc_mid(s1, y1, d, b1, W2)
    s2 = _sc_edge_pass(y2, srci, dsti, zO, D_OUT)
    return _tc_final(s2, y2, d, b2)


# speedup vs baseline: 1.0790x; 1.0364x over previous
import functools

import jax
import jax.numpy as jnp
from jax import lax
from jax.experimental import pallas as pl
from jax.experimental.pallas import tpu as pltpu
from jax.experimental.pallas import tpu_sc as plsc

N = 10000
E = 320000
D_IN = 128
D_H = 64
D_OUT = 32

NC = 2
NS = 16
CH = 128
KD = 80
KE = 160
N_PAD = 10240
RPT = N_PAD // NS
NBUF = 4

_mesh = plsc.VectorSubcoreMesh(core_axis_name="c", subcore_axis_name="s")


def _rsqrt16(x):
    i = lax.bitcast_convert_type(x, jnp.int32)
    i = jnp.int32(0x5F3759DF) - lax.shift_right_arithmetic(i, 1)
    y = lax.bitcast_convert_type(i, jnp.float32)
    xh = x * 0.5
    for _ in range(3):
        y = y * (1.5 - xh * y * y)
    return y


def _sc_degree(dsti, zeros1):
    @functools.partial(
        pl.kernel,
        mesh=_mesh,
        compiler_params=pltpu.CompilerParams(use_tc_tiling_on_sc=False),
        out_type=jax.ShapeDtypeStruct((NC, N_PAD), jnp.float32),
        scratch_types=[
            pltpu.VMEM((KD, CH), jnp.int32),
            pltpu.VMEM((CH,), jnp.float32),
            pltpu.VMEM_SHARED((N_PAD,), jnp.float32),
        ],
    )
    def k(dst_h, z_h, out, dst_v, ones_v, dacc):
        cid = lax.axis_index("c")
        sid = lax.axis_index("s")
        pltpu.sync_copy(dst_h.at[cid, sid], dst_v)
        for t in range(CH // 16):
            ones_v[pl.ds(16 * t, 16)] = jnp.full((16,), 1.0, jnp.float32)
        r0 = sid * RPT
        pltpu.sync_copy(z_h.at[pl.ds(r0, RPT)], dacc.at[pl.ds(r0, RPT)])
        plsc.subcore_barrier()

        def body(j, carry):
            pltpu.sync_copy(ones_v, dacc.at[dst_v.at[j]], add=True)
            return carry

        lax.fori_loop(0, KD, body, 0)
        plsc.subcore_barrier()
        pltpu.sync_copy(dacc.at[pl.ds(r0, RPT)], out.at[cid, pl.ds(r0, RPT)])

    return k(dsti, zeros1)


def _sc_edge_l1(xw, deg2, srci, dsti, zeros2):
    dc = D_H // NC

    @functools.partial(
        pl.kernel,
        mesh=_mesh,
        compiler_params=pltpu.CompilerParams(use_tc_tiling_on_sc=False),
        out_type=jax.ShapeDtypeStruct((N_PAD, D_H), jnp.float32),
        scratch_types=[
            pltpu.VMEM((KE, CH), jnp.int32),
            pltpu.VMEM((KE, CH), jnp.int32),
            [pltpu.VMEM((CH, dc), jnp.float32) for _ in range(NBUF)],
            pltpu.VMEM((RPT, dc), jnp.float32),
            pltpu.VMEM((RPT,), jnp.float32),
            pltpu.VMEM((RPT,), jnp.float32),
            pltpu.VMEM_SHARED((N_PAD, dc), jnp.float32),
            pltpu.VMEM_SHARED((N_PAD, dc), jnp.float32),
            [pltpu.SemaphoreType.DMA for _ in range(NBUF)],
        ],
    )
    def k(xw_h, deg_h, src_h, dst_h, z_h, out, src_v, dst_v, rows,
          xw_v, dg0_v, dg1_v, acc, y_s, sems):
        cid = lax.axis_index("c")
        sid = lax.axis_index("s")
        c0 = cid * dc
        pltpu.sync_copy(src_h.at[sid], src_v)
        pltpu.sync_copy(dst_h.at[sid], dst_v)
        r0 = sid * RPT
        pltpu.sync_copy(z_h.at[pl.ds(r0, RPT)], acc.at[pl.ds(r0, RPT)])
        pltpu.sync_copy(xw_h.at[pl.ds(r0, RPT), pl.ds(c0, dc)], xw_v)
        pltpu.sync_copy(deg_h.at[0, pl.ds(r0, RPT)], dg0_v)
        pltpu.sync_copy(deg_h.at[1, pl.ds(r0, RPT)], dg1_v)

        def scale(g, carry):
            rg = g * 16
            rs = pl.ds(rg, 16)
            d16 = _rsqrt16(dg0_v[rs] + dg1_v[rs] + 1.0)
            for i in range(16):
                dvec = jnp.full((16,), d16[i], jnp.float32)
                for cb in range(dc // 16):
                    cs = pl.ds(cb * 16, 16)
                    xw_v[rg + i, cs] = xw_v[rg + i, cs] * dvec
            return carry

        lax.fori_loop(0, RPT // 16, scale, 0)
        pltpu.sync_copy(xw_v, y_s.at[pl.ds(r0, RPT)])
        plsc.subcore_barrier()

        for b in range(NBUF):
            pltpu.make_async_copy(y_s.at[src_v.at[b]], rows[b], sems[b]).start()

        def body(g, carry):
            j = NBUF * g
            for b in range(NBUF):
                pltpu.make_async_copy(
                    y_s.at[src_v.at[j + b]], rows[b], sems[b]).wait()
                pltpu.sync_copy(rows[b], acc.at[dst_v.at[j + b]], add=True)
                pltpu.make_async_copy(
                    y_s.at[src_v.at[j + b + NBUF]], rows[b], sems[b]).start()
            return carry

        lax.fori_loop(0, KE // NBUF - 1, body, 0)
        for b in range(NBUF):
            j = KE - NBUF + b
            pltpu.make_async_copy(y_s.at[src_v.at[j]], rows[b], sems[b]).wait()
            pltpu.sync_copy(rows[b], acc.at[dst_v.at[j]], add=True)

        plsc.subcore_barrier()
        pltpu.sync_copy(acc.at[pl.ds(r0, RPT)],
                        out.at[pl.ds(r0, RPT), pl.ds(c0, dc)])

    return k(xw, deg2, srci, dsti, zeros2)


def _sc_edge_l2(y2, deg2, srci, dsti, zeros2, b2):
    dc = D_OUT // NC

    @functools.partial(
        pl.kernel,
        mesh=_mesh,
        compiler_params=pltpu.CompilerParams(use_tc_tiling_on_sc=False),
        out_type=jax.ShapeDtypeStruct((N_PAD, D_OUT), jnp.float32),
        scratch_types=[
            pltpu.VMEM((KE, CH), jnp.int32),
            pltpu.VMEM((KE, CH), jnp.int32),
            [pltpu.VMEM((CH, dc), jnp.float32) for _ in range(NBUF)],
            pltpu.VMEM((RPT,), jnp.float32),
            pltpu.VMEM((RPT,), jnp.float32),
            pltpu.VMEM((dc,), jnp.float32),
            pltpu.VMEM_SHARED((N_PAD, dc), jnp.float32),
            pltpu.VMEM_SHARED((N_PAD, dc), jnp.float32),
            [pltpu.SemaphoreType.DMA for _ in range(NBUF)],
        ],
    )
    def k(y_h, deg_h, src_h, dst_h, z_h, b_h, out, src_v, dst_v, rows,
          dg0_v, dg1_v, b_v, acc, y_s, sems):
        cid = lax.axis_index("c")
        sid = lax.axis_index("s")
        c0 = cid * dc
        pltpu.sync_copy(src_h.at[sid], src_v)
        pltpu.sync_copy(dst_h.at[sid], dst_v)
        r0 = sid * RPT
        pltpu.sync_copy(z_h.at[pl.ds(r0, RPT)], acc.at[pl.ds(r0, RPT)])
        pltpu.sync_copy(y_h.at[pl.ds(r0, RPT), pl.ds(c0, dc)],
                        y_s.at[pl.ds(r0, RPT)])
        pltpu.sync_copy(deg_h.at[0, pl.ds(r0, RPT)], dg0_v)
        pltpu.sync_copy(deg_h.at[1, pl.ds(r0, RPT)], dg1_v)
        pltpu.sync_copy(b_h.at[pl.ds(c0, dc)], b_v)
        plsc.subcore_barrier()

        for b in range(NBUF):
            pltpu.make_async_copy(y_s.at[src_v.at[b]], rows[b], sems[b]).start()

        def body(g, carry):
            j = NBUF * g
            for b in range(NBUF):
                pltpu.make_async_copy(
                    y_s.at[src_v.at[j + b]], rows[b], sems[b]).wait()
                pltpu.sync_copy(rows[b], acc.at[dst_v.at[j + b]], add=True)
                pltpu.make_async_copy(
                    y_s.at[src_v.at[j + b + NBUF]], rows[b], sems[b]).start()
            return carry

        lax.fori_loop(0, KE // NBUF - 1, body, 0)
        for b in range(NBUF):
            j = KE - NBUF + b
            pltpu.make_async_copy(y_s.at[src_v.at[j]], rows[b], sems[b]).wait()
            pltpu.sync_copy(rows[b], acc.at[dst_v.at[j]], add=True)

        plsc.subcore_barrier()
        for t in range(RPT // CH):
            ra = r0 + t * CH
            pltpu.sync_copy(acc.at[pl.ds(ra, CH)], rows[0])
            pltpu.sync_copy(y_s.at[pl.ds(ra, CH)], rows[1])

            def erow(g, carry):
                rg = g * 16
                rs = pl.ds(t * CH + rg, 16)
                d16 = _rsqrt16(dg0_v[rs] + dg1_v[rs] + 1.0)
                for i in range(16):
                    dvec = jnp.full((16,), d16[i], jnp.float32)
                    for cb in range(dc // 16):
                        cs = pl.ds(cb * 16, 16)
                        v = (rows[0][rg + i, cs] + rows[1][rg + i, cs]) * dvec
                        rows[0][rg + i, cs] = jnp.maximum(v + b_v[cs], 0.0)
                return carry

            lax.fori_loop(0, CH // 16, erow, 0)
            pltpu.sync_copy(rows[0], out.at[pl.ds(ra, CH), pl.ds(c0, dc)])

    return k(y2, deg2, srci, dsti, zeros2, b2)


def _tc_mm(x_pad, W1):
    def body(x_ref, w_ref, o_ref):
        o_ref[...] = jnp.dot(x_ref[...], w_ref[...],
                             preferred_element_type=jnp.float32)

    return pl.pallas_call(
        body,
        out_shape=jax.ShapeDtypeStruct((N_PAD, D_H), jnp.float32),
    )(x_pad, W1)


def _tc_mid(deg2, s1, xw, b1, W2):
    def body(deg_ref, s_ref, xw_ref, b_ref, w_ref, y2_ref):
        dcol = lax.rsqrt(deg_ref[0] + deg_ref[1] + 1.0)[:, None]
        h = jnp.maximum(
            (s_ref[...] + xw_ref[...] * dcol) * dcol + b_ref[...], 0.0)
        y2_ref[...] = jnp.dot(
            h, w_ref[...], preferred_element_type=jnp.float32) * dcol

    return pl.pallas_call(
        body,
        out_shape=jax.ShapeDtypeStruct((N_PAD, D_OUT), jnp.float32),
    )(deg2, s1, xw, b1.reshape(1, D_H), W2)


def kernel(x, edge_index, W1, b1, W2, b2):
    src = edge_index[0]
    dst = edge_index[1]
    tot = NC * NS * KD * CH
    dst_deg = jnp.concatenate(
        [dst, jnp.full((tot - E,), N, jnp.int32)]).reshape(NC, NS, KD, CH)
    tot_e = NS * KE * CH
    pad_e = tot_e - E
    srci = jnp.concatenate(
        [src, jnp.zeros((pad_e,), jnp.int32)]).reshape(NS, KE, CH)
    dsti = jnp.concatenate(
        [dst, jnp.full((pad_e,), N, jnp.int32)]).reshape(NS, KE, CH)

    x_pad = jnp.pad(x, ((0, N_PAD - N), (0, 0)))
    z1 = jnp.zeros((N_PAD,), jnp.float32)
    zH = jnp.zeros((N_PAD, D_H // NC), jnp.float32)
    zO = jnp.zeros((N_PAD, D_OUT // NC), jnp.float32)

    deg2 = _sc_degree(dst_deg, z1)
    xw = _tc_mm(x_pad, W1)
    s1 = _sc_edge_l1(xw, deg2, srci, dsti, zH)
    y2 = _tc_mid(deg2, s1, xw, b1, W2)
    out = _sc_edge_l2(y2, deg2, srci, dsti, zO, b2)
    return out[:N]


# speedup vs baseline: 1.1770x; 1.0908x over previous
import functools

import jax
import jax.numpy as jnp
from jax import lax
from jax.experimental import pallas as pl
from jax.experimental.pallas import tpu as pltpu
from jax.experimental.pallas import tpu_sc as plsc

N = 10000
E = 320000
D_IN = 128
D_H = 64
D_OUT = 32

NC = 2
NS = 16
CH = 128
NCH = E // CH
KD = NCH // (NC * NS)
KE = NCH // NS
XD = NCH - KD * NC * NS
XE = NCH - KE * NS
N_PAD = 10240
RPT = N_PAD // NS
NBUF = 4

_mesh = plsc.VectorSubcoreMesh(core_axis_name="c", subcore_axis_name="s")


def _rsqrt16(x):
    i = lax.bitcast_convert_type(x, jnp.int32)
    i = jnp.int32(0x5F3759DF) - lax.shift_right_arithmetic(i, 1)
    y = lax.bitcast_convert_type(i, jnp.float32)
    xh = x * 0.5
    for _ in range(3):
        y = y * (1.5 - xh * y * y)
    return y


def _sc_degree(er3, zeros1):
    @functools.partial(
        pl.kernel,
        mesh=_mesh,
        compiler_params=pltpu.CompilerParams(use_tc_tiling_on_sc=False),
        out_type=jax.ShapeDtypeStruct((NC, N_PAD), jnp.float32),
        scratch_types=[
            pltpu.VMEM((KD + 1, CH), jnp.int32),
            pltpu.VMEM((CH,), jnp.float32),
            pltpu.VMEM_SHARED((N_PAD,), jnp.float32),
        ],
    )
    def k(e_h, z_h, out, dst_v, ones_v, dacc):
        cid = lax.axis_index("c")
        sid = lax.axis_index("s")
        w = cid * NS + sid
        start = w * KD + jnp.minimum(w, XD)
        cnt = KD + (w < XD).astype(jnp.int32)
        pltpu.sync_copy(e_h.at[1, pl.ds(start, KD)], dst_v.at[pl.ds(0, KD)])

        @pl.when(w < XD)
        def _():
            pltpu.sync_copy(e_h.at[1, pl.ds(start + KD, 1)],
                            dst_v.at[pl.ds(KD, 1)])
        for t in range(CH // 16):
            ones_v[pl.ds(16 * t, 16)] = jnp.full((16,), 1.0, jnp.float32)
        r0 = sid * RPT
        pltpu.sync_copy(z_h.at[pl.ds(r0, RPT)], dacc.at[pl.ds(r0, RPT)])
        plsc.subcore_barrier()

        def body(j, carry):
            pltpu.sync_copy(ones_v, dacc.at[dst_v.at[j]], add=True)
            return carry

        lax.fori_loop(0, cnt, body, 0)
        plsc.subcore_barrier()
        pltpu.sync_copy(dacc.at[pl.ds(r0, RPT)], out.at[cid, pl.ds(r0, RPT)])

    return k(er3, zeros1)


def _sc_edge_l1(xw, deg2, er3, zeros2):
    dc = D_H // NC

    @functools.partial(
        pl.kernel,
        mesh=_mesh,
        compiler_params=pltpu.CompilerParams(use_tc_tiling_on_sc=False),
        out_type=jax.ShapeDtypeStruct((N_PAD, D_H), jnp.float32),
        scratch_types=[
            pltpu.VMEM((KE + 1, CH), jnp.int32),
            pltpu.VMEM((KE + 1, CH), jnp.int32),
            [pltpu.VMEM((CH, dc), jnp.float32) for _ in range(NBUF)],
            pltpu.VMEM((RPT, dc), jnp.float32),
            pltpu.VMEM((RPT,), jnp.float32),
            pltpu.VMEM((RPT,), jnp.float32),
            pltpu.VMEM_SHARED((N_PAD, dc), jnp.float32),
            pltpu.VMEM_SHARED((N_PAD, dc), jnp.float32),
            [pltpu.SemaphoreType.DMA for _ in range(NBUF)],
        ],
    )
    def k(xw_h, deg_h, e_h, z_h, out, src_v, dst_v, rows,
          xw_v, dg0_v, dg1_v, acc, y_s, sems):
        cid = lax.axis_index("c")
        sid = lax.axis_index("s")
        c0 = cid * dc
        start = sid * KE + jnp.minimum(sid, XE)
        pltpu.sync_copy(e_h.at[0, pl.ds(start, KE)], src_v.at[pl.ds(0, KE)])
        pltpu.sync_copy(e_h.at[1, pl.ds(start, KE)], dst_v.at[pl.ds(0, KE)])

        @pl.when(sid < XE)
        def _():
            pltpu.sync_copy(e_h.at[0, pl.ds(start + KE, 1)],
                            src_v.at[pl.ds(KE, 1)])
            pltpu.sync_copy(e_h.at[1, pl.ds(start + KE, 1)],
                            dst_v.at[pl.ds(KE, 1)])
        r0 = sid * RPT
        pltpu.sync_copy(z_h.at[pl.ds(r0, RPT)], acc.at[pl.ds(r0, RPT)])
        pltpu.sync_copy(xw_h.at[pl.ds(r0, RPT), pl.ds(c0, dc)], xw_v)
        pltpu.sync_copy(deg_h.at[0, pl.ds(r0, RPT)], dg0_v)
        pltpu.sync_copy(deg_h.at[1, pl.ds(r0, RPT)], dg1_v)

        def scale(g, carry):
            rg = g * 16
            rs = pl.ds(rg, 16)
            d16 = _rsqrt16(dg0_v[rs] + dg1_v[rs] + 1.0)
            for i in range(16):
                dvec = jnp.full((16,), d16[i], jnp.float32)
                for cb in range(dc // 16):
                    cs = pl.ds(cb * 16, 16)
                    xw_v[rg + i, cs] = xw_v[rg + i, cs] * dvec
            return carry

        lax.fori_loop(0, RPT // 16, scale, 0)
        pltpu.sync_copy(xw_v, y_s.at[pl.ds(r0, RPT)])
        plsc.subcore_barrier()

        for b in range(NBUF):
            pltpu.make_async_copy(y_s.at[src_v.at[b]], rows[b], sems[b]).start()

        def body(g, carry):
            j = NBUF * g
            for b in range(NBUF):
                pltpu.make_async_copy(
                    y_s.at[src_v.at[j + b]], rows[b], sems[b]).wait()
                pltpu.sync_copy(rows[b], acc.at[dst_v.at[j + b]], add=True)
                pltpu.make_async_copy(
                    y_s.at[src_v.at[j + b + NBUF]], rows[b], sems[b]).start()
            return carry

        lax.fori_loop(0, KE // NBUF - 1, body, 0)
        for b in range(NBUF):
            j = KE - NBUF + b
            pltpu.make_async_copy(y_s.at[src_v.at[j]], rows[b], sems[b]).wait()
            pltpu.sync_copy(rows[b], acc.at[dst_v.at[j]], add=True)

        @pl.when(sid < XE)
        def _():
            pltpu.make_async_copy(
                y_s.at[src_v.at[KE]], rows[0], sems[0]).start()
            pltpu.make_async_copy(
                y_s.at[src_v.at[KE]], rows[0], sems[0]).wait()
            pltpu.sync_copy(rows[0], acc.at[dst_v.at[KE]], add=True)

        plsc.subcore_barrier()
        pltpu.sync_copy(acc.at[pl.ds(r0, RPT)],
                        out.at[pl.ds(r0, RPT), pl.ds(c0, dc)])

    return k(xw, deg2, er3, zeros2)


def _sc_edge_l2(y2, deg2, er3, zeros2, b2):
    dc = D_OUT // NC

    @functools.partial(
        pl.kernel,
        mesh=_mesh,
        compiler_params=pltpu.CompilerParams(use_tc_tiling_on_sc=False),
        out_type=jax.ShapeDtypeStruct((N_PAD, D_OUT), jnp.float32),
        scratch_types=[
            pltpu.VMEM((KE + 1, CH), jnp.int32),
            pltpu.VMEM((KE + 1, CH), jnp.int32),
            [pltpu.VMEM((CH, dc), jnp.float32) for _ in range(NBUF)],
            pltpu.VMEM((RPT,), jnp.float32),
            pltpu.VMEM((RPT,), jnp.float32),
            pltpu.VMEM((dc,), jnp.float32),
            pltpu.VMEM_SHARED((N_PAD, dc), jnp.float32),
            pltpu.VMEM_SHARED((N_PAD, dc), jnp.float32),
            [pltpu.SemaphoreType.DMA for _ in range(NBUF)],
        ],
    )
    def k(y_h, deg_h, e_h, z_h, b_h, out, src_v, dst_v, rows,
          dg0_v, dg1_v, b_v, acc, y_s, sems):
        cid = lax.axis_index("c")
        sid = lax.axis_index("s")
        c0 = cid * dc
        start = sid * KE + jnp.minimum(sid, XE)
        pltpu.sync_copy(e_h.at[0, pl.ds(start, KE)], src_v.at[pl.ds(0, KE)])
        pltpu.sync_copy(e_h.at[1, pl.ds(start, KE)], dst_v.at[pl.ds(0, KE)])

        @pl.when(sid < XE)
        def _():
            pltpu.sync_copy(e_h.at[0, pl.ds(start + KE, 1)],
                            src_v.at[pl.ds(KE, 1)])
            pltpu.sync_copy(e_h.at[1, pl.ds(start + KE, 1)],
                            dst_v.at[pl.ds(KE, 1)])
        r0 = sid * RPT
        pltpu.sync_copy(z_h.at[pl.ds(r0, RPT)], acc.at[pl.ds(r0, RPT)])
        pltpu.sync_copy(y_h.at[pl.ds(r0, RPT), pl.ds(c0, dc)],
                        y_s.at[pl.ds(r0, RPT)])
        pltpu.sync_copy(deg_h.at[0, pl.ds(r0, RPT)], dg0_v)
        pltpu.sync_copy(deg_h.at[1, pl.ds(r0, RPT)], dg1_v)
        pltpu.sync_copy(b_h.at[pl.ds(c0, dc)], b_v)
        plsc.subcore_barrier()

        for b in range(NBUF):
            pltpu.make_async_copy(y_s.at[src_v.at[b]], rows[b], sems[b]).start()

        def body(g, carry):
            j = NBUF * g
            for b in range(NBUF):
                pltpu.make_async_copy(
                    y_s.at[src_v.at[j + b]], rows[b], sems[b]).wait()
                pltpu.sync_copy(rows[b], acc.at[dst_v.at[j + b]], add=True)
                pltpu.make_async_copy(
                    y_s.at[src_v.at[j + b + NBUF]], rows[b], sems[b]).start()
            return carry

        lax.fori_loop(0, KE // NBUF - 1, body, 0)
        for b in range(NBUF):
            j = KE - NBUF + b
            pltpu.make_async_copy(y_s.at[src_v.at[j]], rows[b], sems[b]).wait()
            pltpu.sync_copy(rows[b], acc.at[dst_v.at[j]], add=True)

        @pl.when(sid < XE)
        def _():
            pltpu.make_async_copy(
                y_s.at[src_v.at[KE]], rows[0], sems[0]).start()
            pltpu.make_async_copy(
                y_s.at[src_v.at[KE]], rows[0], sems[0]).wait()
            pltpu.sync_copy(rows[0], acc.at[dst_v.at[KE]], add=True)

        plsc.subcore_barrier()
        for t in range(RPT // CH):
            ra = r0 + t * CH
            pltpu.sync_copy(acc.at[pl.ds(ra, CH)], rows[0])
            pltpu.sync_copy(y_s.at[pl.ds(ra, CH)], rows[1])

            def erow(g, carry):
                rg = g * 16
                rs = pl.ds(t * CH + rg, 16)
                d16 = _rsqrt16(dg0_v[rs] + dg1_v[rs] + 1.0)
                for i in range(16):
                    dvec = jnp.full((16,), d16[i], jnp.float32)
                    for cb in range(dc // 16):
                        cs = pl.ds(cb * 16, 16)
                        v = (rows[0][rg + i, cs] + rows[1][rg + i, cs]) * dvec
                        rows[0][rg + i, cs] = jnp.maximum(v + b_v[cs], 0.0)
                return carry

            lax.fori_loop(0, CH // 16, erow, 0)
            pltpu.sync_copy(rows[0], out.at[pl.ds(ra, CH), pl.ds(c0, dc)])

    return k(y2, deg2, er3, zeros2, b2)


def _tc_mm(x_pad, W1):
    def body(x_ref, w_ref, o_ref):
        o_ref[...] = jnp.dot(x_ref[...], w_ref[...],
                             preferred_element_type=jnp.float32)

    return pl.pallas_call(
        body,
        out_shape=jax.ShapeDtypeStruct((N_PAD, D_H), jnp.float32),
    )(x_pad, W1)


def _tc_mid(deg2, s1, xw, b1, W2):
    def body(deg_ref, s_ref, xw_ref, b_ref, w_ref, y2_ref):
        dcol = lax.rsqrt(deg_ref[0] + deg_ref[1] + 1.0)[:, None]
        h = jnp.maximum(
            (s_ref[...] + xw_ref[...] * dcol) * dcol + b_ref[...], 0.0)
        y2_ref[...] = jnp.dot(
            h, w_ref[...], preferred_element_type=jnp.float32) * dcol

    return pl.pallas_call(
        body,
        out_shape=jax.ShapeDtypeStruct((N_PAD, D_OUT), jnp.float32),
    )(deg2, s1, xw, b1.reshape(1, D_H), W2)


def kernel(x, edge_index, W1, b1, W2, b2):
    er3 = edge_index.reshape(2, NCH, CH)

    x_pad = jnp.pad(x, ((0, N_PAD - N), (0, 0)))
    z1 = jnp.zeros((N_PAD,), jnp.float32)
    zH = jnp.zeros((N_PAD, D_H // NC), jnp.float32)
    zO = jnp.zeros((N_PAD, D_OUT // NC), jnp.float32)

    deg2 = _sc_degree(er3, z1)
    xw = _tc_mm(x_pad, W1)
    s1 = _sc_edge_l1(xw, deg2, er3, zH)
    y2 = _tc_mid(deg2, s1, xw, b1, W2)
    out = _sc_edge_l2(y2, deg2, er3, zO, b2)
    return out[:N]
